# baseline (speedup 1.0000x reference)
import functools

import jax
import jax.numpy as jnp
from jax import lax
from jax.experimental import pallas as pl
from jax.experimental.pallas import tpu as pltpu
from jax.experimental.pallas import tpu_sc as plsc

N = 10000
NPAD = 10112
D = 128
NC, NS, L = 2, 16, 16
NW = NC * NS
CHUNK = 128
E_TOT = 320000 + N
EPW = 10752
E_PAD = NW * EPW
NCHUNKS = EPW // CHUNK
RPT = NPAD // NS
ZR = RPT // 2

_MESH = dict(core_axis_name="c", subcore_axis_name="s")


@functools.partial(
    pl.kernel,
    out_type=jax.ShapeDtypeStruct((NC, NPAD, L), jnp.float32),
    mesh=plsc.VectorSubcoreMesh(**_MESH),
    scratch_types=[
        pltpu.VMEM((CHUNK,), jnp.int32),
        pltpu.VMEM((CHUNK, L), jnp.float32),
        pltpu.VMEM((ZR, L), jnp.float32),
        pltpu.VMEM_SHARED((NPAD, L), jnp.float32),
        pltpu.SemaphoreType.DMA,
    ],
    compiler_params=pltpu.CompilerParams(use_tc_tiling_on_sc=False),
)
def _deg_kernel(dst_hbm, out_hbm, idx_v, ones_v, zero_v, acc, sem):
    c = lax.axis_index("c")
    s = lax.axis_index("s")

    def fill(i, _):
        ones_v[i] = jnp.ones((L,), jnp.float32)
        return 0

    lax.fori_loop(0, CHUNK, fill, 0)

    def zfill(i, _):
        zero_v[i] = jnp.zeros((L,), jnp.float32)
        return 0

    lax.fori_loop(0, ZR, zfill, 0)
    pltpu.sync_copy(zero_v, acc.at[pl.ds(s * RPT, ZR)])
    pltpu.sync_copy(zero_v, acc.at[pl.ds(s * RPT + ZR, ZR)])
    plsc.subcore_barrier()

    base_w = (c * NS + s) * EPW

    def chunk_body(t, _):
        pltpu.sync_copy(dst_hbm.at[pl.ds(base_w + t * CHUNK, CHUNK)], idx_v)
        pltpu.sync_copy(ones_v, acc.at[idx_v], add=True)
        return 0

    lax.fori_loop(0, NCHUNKS, chunk_body, 0)
    plsc.subcore_barrier()
    pltpu.sync_copy(acc.at[pl.ds(s * RPT, RPT)],
                    out_hbm.at[c, pl.ds(s * RPT, RPT)])


HALF = D // 2


@functools.partial(
    pl.kernel,
    out_type=jax.ShapeDtypeStruct((NC, 2, NPAD, HALF), jnp.float32),
    mesh=plsc.VectorSubcoreMesh(**_MESH),
    scratch_types=[
        pltpu.VMEM((NCHUNKS, CHUNK), jnp.int32),
        pltpu.VMEM((NCHUNKS, CHUNK), jnp.int32),
        [pltpu.VMEM((CHUNK, HALF), jnp.float32) for _ in range(4)],
        pltpu.VMEM((ZR, HALF), jnp.float32),
        pltpu.VMEM_SHARED((NPAD, HALF), jnp.float32),
        [pltpu.SemaphoreType.DMA for _ in range(2)],
        [pltpu.SemaphoreType.DMA for _ in range(4)],
    ],
    compiler_params=pltpu.CompilerParams(use_tc_tiling_on_sc=False),
)
def _edge_kernel(src0_hbm, src1_hbm, dst_hbm, gv_hbm, out_hbm,
                 si_all, di_all, rows, zero_v, acc, gsems, ssems):
    c = lax.axis_index("c")
    s = lax.axis_index("s")
    w = c * NS + s

    def zfill(i, _):
        zero_v[i // 4, pl.ds((i % 4) * L, L)] = jnp.zeros((L,), jnp.float32)
        return 0

    lax.fori_loop(0, ZR * (HALF // L), zfill, 0)
    pltpu.sync_copy(dst_hbm.at[w], di_all)

    for h, src_hbm in ((0, src0_hbm), (1, src1_hbm)):
        pltpu.sync_copy(zero_v, acc.at[pl.ds(s * RPT, ZR)])
        pltpu.sync_copy(zero_v, acc.at[pl.ds(s * RPT + ZR, ZR)])
        pltpu.sync_copy(src_hbm.at[w], si_all)
        plsc.subcore_barrier()

        pltpu.async_copy(gv_hbm.at[si_all.at[0]], rows[0], gsems[0])

        @pl.loop(0, NCHUNKS, step=2)
        def chunk_pair(t0):
            for b in range(2):
                t = t0 + b

                @pl.when(t + 1 < NCHUNKS)
                def _():
                    pltpu.async_copy(gv_hbm.at[si_all.at[t + 1]],
                                     rows[1 - b], gsems[1 - b])

                pltpu.make_async_copy(gv_hbm.at[si_all.at[t]], rows[b],
                                      gsems[b]).wait()
                pltpu.sync_copy(rows[b], acc.at[di_all.at[t]], add=True)

        plsc.subcore_barrier()
        pltpu.sync_copy(acc.at[pl.ds(s * RPT, RPT)],
                        out_hbm.at[c, h, pl.ds(s * RPT, RPT)])


BR = 2528
GRID = NPAD // BR


def _dinv_block(d0_ref, d1_ref):
    deg = d0_ref[:, :1] + d1_ref[:, :1]
    return jnp.where(deg > 0, lax.rsqrt(deg), 0.0)


def _tc_first_body(x_ref, w_ref, d0_ref, d1_ref, g_ref):
    dinv = _dinv_block(d0_ref, d1_ref)
    g_ref[...] = dinv * jnp.dot(x_ref[...], w_ref[...],
                                preferred_element_type=jnp.float32)


def _combine(p00, p10, p01, p11, d0_ref, d1_ref, b_ref):
    dinv = _dinv_block(d0_ref, d1_ref)
    h0 = jnp.maximum(dinv * (p00[...] + p10[...]) + b_ref[:, :HALF], 0.0)
    h1 = jnp.maximum(dinv * (p01[...] + p11[...]) + b_ref[:, HALF:], 0.0)
    return dinv, jnp.concatenate([h0, h1], axis=1)


def _tc_mid_body(p00, p10, p01, p11, d0_ref, d1_ref, b_ref, w_ref, g_ref):
    dinv, h = _combine(p00, p10, p01, p11, d0_ref, d1_ref, b_ref)
    g_ref[...] = dinv * jnp.dot(h, w_ref[...],
                                preferred_element_type=jnp.float32)


def _tc_last_body(p00, p10, p01, p11, d0_ref, d1_ref, b_ref, w_ref, bb_ref,
                  o_ref):
    _, h = _combine(p00, p10, p01, p11, d0_ref, d1_ref, b_ref)
    o_ref[...] = jnp.dot(h, w_ref[...],
                         preferred_element_type=jnp.float32) + bb_ref[...]


_ROWS = pl.BlockSpec((BR, D), lambda i: (i, 0))
_HROWS = pl.BlockSpec((BR, HALF), lambda i: (i, 0))
_MAT = pl.BlockSpec((D, D), lambda i: (0, 0))
_DEG = pl.BlockSpec((BR, L), lambda i: (i, 0))
_VEC = pl.BlockSpec((1, D), lambda i: (0, 0))
_OUT = jax.ShapeDtypeStruct((NPAD, D), jnp.float32)

_tc_first = pl.pallas_call(
    _tc_first_body, grid=(GRID,),
    in_specs=[_ROWS, _MAT, _DEG, _DEG], out_specs=_ROWS, out_shape=_OUT)

_tc_mid = pl.pallas_call(
    _tc_mid_body, grid=(GRID,),
    in_specs=[_HROWS, _HROWS, _HROWS, _HROWS, _DEG, _DEG, _VEC, _MAT],
    out_specs=_ROWS, out_shape=_OUT)

_tc_last = pl.pallas_call(
    _tc_last_body, grid=(GRID,),
    in_specs=[_HROWS, _HROWS, _HROWS, _HROWS, _DEG, _DEG, _VEC, _MAT, _VEC],
    out_specs=_ROWS, out_shape=_OUT)


def kernel(x, edge_index, W0, b0, W1, b1, W2, b2, fcW, fcb):
    ei = edge_index.astype(jnp.int32)
    loop = jnp.arange(N, dtype=jnp.int32)
    npad_e = E_PAD - E_TOT
    src = jnp.concatenate([ei[0], loop, jnp.zeros((npad_e,), jnp.int32)])
    pad_dst = N + (jnp.arange(npad_e, dtype=jnp.int32) % (NPAD - N))
    dst = jnp.concatenate([ei[1], loop, pad_dst])
    src0 = src * 2
    src1 = src0 + 1
    xp = jnp.pad(x, ((0, NPAD - N), (0, 0)))

    degw = _deg_kernel(dst)
    d0, d1 = degw[0], degw[1]

    def halves(g):
        return g.reshape(2 * NPAD, HALF)

    src0_3 = src0.reshape(NW, NCHUNKS, CHUNK)
    src1_3 = src1.reshape(NW, NCHUNKS, CHUNK)
    dst3 = dst.reshape(NW, NCHUNKS, CHUNK)

    g0 = _tc_first(xp, W0, d0, d1)
    a0 = _edge_kernel(src0_3, src1_3, dst3, halves(g0))
    g1 = _tc_mid(a0[0, 0], a0[1, 0], a0[0, 1], a0[1, 1], d0, d1,
                 b0.reshape(1, D), W1)
    a1 = _edge_kernel(src0_3, src1_3, dst3, halves(g1))
    g2 = _tc_mid(a1[0, 0], a1[1, 0], a1[0, 1], a1[1, 1], d0, d1,
                 b1.reshape(1, D), W2)
    a2 = _edge_kernel(src0_3, src1_3, dst3, halves(g2))
    out = _tc_last(a2[0, 0], a2[1, 0], a2[0, 1], a2[1, 1], d0, d1,
                   b2.reshape(1, D), fcW, fcb.reshape(1, D))
    return out[:N]


# speedup vs baseline: 3.5972x; 3.5972x over previous
import functools

import jax
import jax.numpy as jnp
from jax import lax
from jax.experimental import pallas as pl
from jax.experimental.pallas import tpu as pltpu
from jax.experimental.pallas import tpu_sc as plsc

N = 10000
NPAD = 10112
D = 128
NC, NS, L = 2, 16, 16
NW = NC * NS
CHUNK = 128
E_TOT = 320000 + N
EPW = 10496
E_PAD = NW * EPW
NCHUNKS = EPW // CHUNK
RPT = NPAD // NS
ZR = RPT // 2

_MESH = dict(core_axis_name="c", subcore_axis_name="s")


@functools.partial(
    pl.kernel,
    out_type=jax.ShapeDtypeStruct((NC, NPAD, L), jnp.float32),
    mesh=plsc.VectorSubcoreMesh(**_MESH),
    scratch_types=[
        pltpu.VMEM((CHUNK,), jnp.int32),
        pltpu.VMEM((CHUNK, L), jnp.float32),
        pltpu.VMEM((ZR, L), jnp.float32),
        pltpu.VMEM_SHARED((NPAD, L), jnp.float32),
        pltpu.SemaphoreType.DMA,
    ],
    compiler_params=pltpu.CompilerParams(use_tc_tiling_on_sc=False),
)
def _deg_kernel(dst_hbm, out_hbm, idx_v, ones_v, zero_v, acc, sem):
    c = lax.axis_index("c")
    s = lax.axis_index("s")

    def fill(i, _):
        ones_v[i] = jnp.ones((L,), jnp.float32)
        return 0

    lax.fori_loop(0, CHUNK, fill, 0)

    def zfill(i, _):
        zero_v[i] = jnp.zeros((L,), jnp.float32)
        return 0

    lax.fori_loop(0, ZR, zfill, 0)
    pltpu.sync_copy(zero_v, acc.at[pl.ds(s * RPT, ZR)])
    pltpu.sync_copy(zero_v, acc.at[pl.ds(s * RPT + ZR, ZR)])
    plsc.subcore_barrier()

    base_w = (c * NS + s) * EPW

    def chunk_body(t, _):
        pltpu.sync_copy(dst_hbm.at[pl.ds(base_w + t * CHUNK, CHUNK)], idx_v)
        pltpu.sync_copy(ones_v, acc.at[idx_v], add=True)
        return 0

    lax.fori_loop(0, NCHUNKS, chunk_body, 0)
    plsc.subcore_barrier()
    pltpu.sync_copy(acc.at[pl.ds(s * RPT, RPT)],
                    out_hbm.at[c, pl.ds(s * RPT, RPT)])


HALF = D // 2


@functools.partial(
    pl.kernel,
    out_type=jax.ShapeDtypeStruct((NC, 2, NPAD, HALF), jnp.float32),
    mesh=plsc.VectorSubcoreMesh(**_MESH),
    scratch_types=[
        pltpu.VMEM((NCHUNKS, CHUNK), jnp.int32),
        pltpu.VMEM((NCHUNKS, CHUNK), jnp.int32),
        [pltpu.VMEM((CHUNK, HALF), jnp.float32) for _ in range(4)],
        pltpu.VMEM((ZR, HALF), jnp.float32),
        pltpu.VMEM_SHARED((NPAD, HALF), jnp.float32),
        [pltpu.SemaphoreType.DMA for _ in range(2)],
        [pltpu.SemaphoreType.DMA for _ in range(4)],
    ],
    compiler_params=pltpu.CompilerParams(use_tc_tiling_on_sc=False),
)
def _edge_kernel(src0_hbm, src1_hbm, dst_hbm, gv_hbm, out_hbm,
                 si_all, di_all, rows, zero_v, acc, gsems, ssems):
    c = lax.axis_index("c")
    s = lax.axis_index("s")
    w = c * NS + s

    def zfill(i, _):
        zero_v[i // 4, pl.ds((i % 4) * L, L)] = jnp.zeros((L,), jnp.float32)
        return 0

    lax.fori_loop(0, ZR * (HALF // L), zfill, 0)
    pltpu.sync_copy(dst_hbm.at[w], di_all)

    for h, src_hbm in ((0, src0_hbm), (1, src1_hbm)):
        pltpu.sync_copy(zero_v, acc.at[pl.ds(s * RPT, ZR)])
        pltpu.sync_copy(zero_v, acc.at[pl.ds(s * RPT + ZR, ZR)])
        pltpu.sync_copy(src_hbm.at[w], si_all)
        plsc.subcore_barrier()

        pltpu.async_copy(gv_hbm.at[si_all.at[0]], rows[0], gsems[0])

        @pl.loop(0, NCHUNKS, step=2)
        def chunk_pair(t0):
            for b in range(2):
                t = t0 + b

                @pl.when(t + 1 < NCHUNKS)
                def _():
                    pltpu.async_copy(gv_hbm.at[si_all.at[t + 1]],
                                     rows[1 - b], gsems[1 - b])

                pltpu.make_async_copy(gv_hbm.at[si_all.at[t]], rows[b],
                                      gsems[b]).wait()
                pltpu.sync_copy(rows[b], acc.at[di_all.at[t]], add=True)

        plsc.subcore_barrier()
        pltpu.sync_copy(acc.at[pl.ds(s * RPT, RPT)],
                        out_hbm.at[c, h, pl.ds(s * RPT, RPT)])


BR = 2528
GRID = NPAD // BR


def _dinv_block(d0_ref, d1_ref):
    deg = d0_ref[:, :1] + d1_ref[:, :1]
    return jnp.where(deg > 0, lax.rsqrt(deg), 0.0)


def _tc_first_body(x_ref, w_ref, d0_ref, d1_ref, g_ref):
    dinv = _dinv_block(d0_ref, d1_ref)
    g_ref[...] = dinv * jnp.dot(x_ref[...], w_ref[...],
                                preferred_element_type=jnp.float32)


def _combine(p00, p10, p01, p11, d0_ref, d1_ref, b_ref):
    dinv = _dinv_block(d0_ref, d1_ref)
    h0 = jnp.maximum(dinv * (p00[...] + p10[...]) + b_ref[:, :HALF], 0.0)
    h1 = jnp.maximum(dinv * (p01[...] + p11[...]) + b_ref[:, HALF:], 0.0)
    return dinv, jnp.concatenate([h0, h1], axis=1)


def _tc_mid_body(p00, p10, p01, p11, d0_ref, d1_ref, b_ref, w_ref, g_ref):
    dinv, h = _combine(p00, p10, p01, p11, d0_ref, d1_ref, b_ref)
    g_ref[...] = dinv * jnp.dot(h, w_ref[...],
                                preferred_element_type=jnp.float32)


def _tc_last_body(p00, p10, p01, p11, d0_ref, d1_ref, b_ref, w_ref, bb_ref,
                  o_ref):
    _, h = _combine(p00, p10, p01, p11, d0_ref, d1_ref, b_ref)
    o_ref[...] = jnp.dot(h, w_ref[...],
                         preferred_element_type=jnp.float32) + bb_ref[...]


_ROWS = pl.BlockSpec((BR, D), lambda i: (i, 0))
_HROWS = pl.BlockSpec((BR, HALF), lambda i: (i, 0))
_MAT = pl.BlockSpec((D, D), lambda i: (0, 0))
_DEG = pl.BlockSpec((BR, L), lambda i: (i, 0))
_VEC = pl.BlockSpec((1, D), lambda i: (0, 0))
_OUT = jax.ShapeDtypeStruct((NPAD, D), jnp.float32)

_tc_first = pl.pallas_call(
    _tc_first_body, grid=(GRID,),
    in_specs=[_ROWS, _MAT, _DEG, _DEG], out_specs=_ROWS, out_shape=_OUT)

_tc_mid = pl.pallas_call(
    _tc_mid_body, grid=(GRID,),
    in_specs=[_HROWS, _HROWS, _HROWS, _HROWS, _DEG, _DEG, _VEC, _MAT],
    out_specs=_ROWS, out_shape=_OUT)

_tc_last = pl.pallas_call(
    _tc_last_body, grid=(GRID,),
    in_specs=[_HROWS, _HROWS, _HROWS, _HROWS, _DEG, _DEG, _VEC, _MAT, _VEC],
    out_specs=_ROWS, out_shape=_OUT)


def kernel(x, edge_index, W0, b0, W1, b1, W2, b2, fcW, fcb):
    ei = edge_index.astype(jnp.int32)
    loop = jnp.arange(N, dtype=jnp.int32)
    npad_e = E_PAD - E_TOT
    pad_src = jnp.arange(npad_e, dtype=jnp.int32) % N
    src = jnp.concatenate([ei[0], loop, pad_src])
    pad_dst = N + (jnp.arange(npad_e, dtype=jnp.int32) % (NPAD - N))
    dst = jnp.concatenate([ei[1], loop, pad_dst])
    src0 = src * 2
    src1 = src0 + 1
    xp = jnp.pad(x, ((0, NPAD - N), (0, 0)))

    degw = _deg_kernel(dst)
    d0, d1 = degw[0], degw[1]

    def halves(g):
        return g.reshape(2 * NPAD, HALF)

    src0_3 = src0.reshape(NW, NCHUNKS, CHUNK)
    src1_3 = src1.reshape(NW, NCHUNKS, CHUNK)
    dst3 = dst.reshape(NW, NCHUNKS, CHUNK)

    g0 = _tc_first(xp, W0, d0, d1)
    a0 = _edge_kernel(src0_3, src1_3, dst3, halves(g0))
    g1 = _tc_mid(a0[0, 0], a0[1, 0], a0[0, 1], a0[1, 1], d0, d1,
                 b0.reshape(1, D), W1)
    a1 = _edge_kernel(src0_3, src1_3, dst3, halves(g1))
    g2 = _tc_mid(a1[0, 0], a1[1, 0], a1[0, 1], a1[1, 1], d0, d1,
                 b1.reshape(1, D), W2)
    a2 = _edge_kernel(src0_3, src1_3, dst3, halves(g2))
    out = _tc_last(a2[0, 0], a2[1, 0], a2[0, 1], a2[1, 1], d0, d1,
                   b2.reshape(1, D), fcW, fcb.reshape(1, D))
    return out[:N]


# speedup vs baseline: 3.6559x; 1.0163x over previous
import functools

import jax
import jax.numpy as jnp
from jax import lax
from jax.experimental import pallas as pl
from jax.experimental.pallas import tpu as pltpu
from jax.experimental.pallas import tpu_sc as plsc

N = 10000
NPAD = 10112
D = 128
NC, NS, L = 2, 16, 16
NW = NC * NS
CHUNK = 128
E_TOT = 320000 + N
EPW = 10752
E_PAD = NW * EPW
NCHUNKS = EPW // CHUNK
RPT = NPAD // NS
ZR = RPT // 2

_MESH = dict(core_axis_name="c", subcore_axis_name="s")


@functools.partial(
    pl.kernel,
    out_type=jax.ShapeDtypeStruct((NC, NPAD, L), jnp.float32),
    mesh=plsc.VectorSubcoreMesh(**_MESH),
    scratch_types=[
        pltpu.VMEM((CHUNK,), jnp.int32),
        pltpu.VMEM((CHUNK, L), jnp.float32),
        pltpu.VMEM((ZR, L), jnp.float32),
        pltpu.VMEM_SHARED((NPAD, L), jnp.float32),
        pltpu.SemaphoreType.DMA,
    ],
    compiler_params=pltpu.CompilerParams(use_tc_tiling_on_sc=False),
)
def _deg_kernel(dst_hbm, out_hbm, idx_v, ones_v, zero_v, acc, sem):
    c = lax.axis_index("c")
    s = lax.axis_index("s")

    def fill(i, _):
        ones_v[i] = jnp.ones((L,), jnp.float32)
        return 0

    lax.fori_loop(0, CHUNK, fill, 0)

    def zfill(i, _):
        zero_v[i] = jnp.zeros((L,), jnp.float32)
        return 0

    lax.fori_loop(0, ZR, zfill, 0)
    pltpu.sync_copy(zero_v, acc.at[pl.ds(s * RPT, ZR)])
    pltpu.sync_copy(zero_v, acc.at[pl.ds(s * RPT + ZR, ZR)])
    plsc.subcore_barrier()

    base_w = (c * NS + s) * EPW

    def chunk_body(t, _):
        pltpu.sync_copy(dst_hbm.at[pl.ds(base_w + t * CHUNK, CHUNK)], idx_v)
        pltpu.sync_copy(ones_v, acc.at[idx_v], add=True)
        return 0

    lax.fori_loop(0, NCHUNKS, chunk_body, 0)
    plsc.subcore_barrier()
    pltpu.sync_copy(acc.at[pl.ds(s * RPT, RPT)],
                    out_hbm.at[c, pl.ds(s * RPT, RPT)])


HALF = D // 2


@functools.partial(
    pl.kernel,
    out_type=jax.ShapeDtypeStruct((NC, 2, NPAD, HALF), jnp.float32),
    mesh=plsc.VectorSubcoreMesh(**_MESH),
    scratch_types=[
        pltpu.VMEM((NCHUNKS, CHUNK), jnp.int32),
        pltpu.VMEM((NCHUNKS, CHUNK), jnp.int32),
        [pltpu.VMEM((CHUNK, HALF), jnp.float32) for _ in range(4)],
        pltpu.VMEM((ZR, HALF), jnp.float32),
        pltpu.VMEM_SHARED((NPAD, HALF), jnp.float32),
        [pltpu.SemaphoreType.DMA for _ in range(4)],
        [pltpu.SemaphoreType.DMA for _ in range(4)],
    ],
    compiler_params=pltpu.CompilerParams(use_tc_tiling_on_sc=False),
)
def _edge_kernel(src0_hbm, src1_hbm, dst_hbm, gv_hbm, out_hbm,
                 si_all, di_all, rows, zero_v, acc, gsems, ssems):
    c = lax.axis_index("c")
    s = lax.axis_index("s")
    w = c * NS + s

    def zfill(i, _):
        zero_v[i // 4, pl.ds((i % 4) * L, L)] = jnp.zeros((L,), jnp.float32)
        return 0

    lax.fori_loop(0, ZR * (HALF // L), zfill, 0)
    pltpu.sync_copy(dst_hbm.at[w], di_all)

    for h, src_hbm in ((0, src0_hbm), (1, src1_hbm)):
        pltpu.sync_copy(zero_v, acc.at[pl.ds(s * RPT, ZR)])
        pltpu.sync_copy(zero_v, acc.at[pl.ds(s * RPT + ZR, ZR)])
        pltpu.sync_copy(src_hbm.at[w], si_all)
        plsc.subcore_barrier()

        pltpu.async_copy(gv_hbm.at[si_all.at[0]], rows[0], gsems[0])
        pltpu.async_copy(gv_hbm.at[si_all.at[1]], rows[1], gsems[1])

        @pl.loop(0, NCHUNKS, step=4)
        def chunk_quad(t0):
            for b in range(4):
                t = t0 + b
                nb = (b + 2) % 4
                pltpu.make_async_copy(gv_hbm.at[si_all.at[t]], rows[b],
                                      gsems[b]).wait()
                pltpu.async_copy(rows[b], acc.at[di_all.at[t]], ssems[b],
                                 add=True)

                @pl.when((t >= 2) & (t + 2 < NCHUNKS))
                def _():
                    pltpu.make_async_copy(rows[nb], acc.at[di_all.at[t]],
                                          ssems[nb]).wait()

                @pl.when(t + 2 < NCHUNKS)
                def _():
                    pltpu.async_copy(gv_hbm.at[si_all.at[t + 2]], rows[nb],
                                     gsems[nb])

        for b in range(4):
            pltpu.make_async_copy(rows[b], acc.at[di_all.at[0]],
                                  ssems[b]).wait()
        plsc.subcore_barrier()
        pltpu.sync_copy(acc.at[pl.ds(s * RPT, RPT)],
                        out_hbm.at[c, h, pl.ds(s * RPT, RPT)])


BR = 2528
GRID = NPAD // BR


def _dinv_block(d0_ref, d1_ref):
    deg = d0_ref[:, :1] + d1_ref[:, :1]
    return jnp.where(deg > 0, lax.rsqrt(deg), 0.0)


def _tc_first_body(x_ref, w_ref, d0_ref, d1_ref, g_ref):
    dinv = _dinv_block(d0_ref, d1_ref)
    g_ref[...] = dinv * jnp.dot(x_ref[...], w_ref[...],
                                preferred_element_type=jnp.float32)


def _combine(p00, p10, p01, p11, d0_ref, d1_ref, b_ref):
    dinv = _dinv_block(d0_ref, d1_ref)
    h0 = jnp.maximum(dinv * (p00[...] + p10[...]) + b_ref[:, :HALF], 0.0)
    h1 = jnp.maximum(dinv * (p01[...] + p11[...]) + b_ref[:, HALF:], 0.0)
    return dinv, jnp.concatenate([h0, h1], axis=1)


def _tc_mid_body(p00, p10, p01, p11, d0_ref, d1_ref, b_ref, w_ref, g_ref):
    dinv, h = _combine(p00, p10, p01, p11, d0_ref, d1_ref, b_ref)
    g_ref[...] = dinv * jnp.dot(h, w_ref[...],
                                preferred_element_type=jnp.float32)


def _tc_last_body(p00, p10, p01, p11, d0_ref, d1_ref, b_ref, w_ref, bb_ref,
                  o_ref):
    _, h = _combine(p00, p10, p01, p11, d0_ref, d1_ref, b_ref)
    o_ref[...] = jnp.dot(h, w_ref[...],
                         preferred_element_type=jnp.float32) + bb_ref[...]


_ROWS = pl.BlockSpec((BR, D), lambda i: (i, 0))
_HROWS = pl.BlockSpec((BR, HALF), lambda i: (i, 0))
_MAT = pl.BlockSpec((D, D), lambda i: (0, 0))
_DEG = pl.BlockSpec((BR, L), lambda i: (i, 0))
_VEC = pl.BlockSpec((1, D), lambda i: (0, 0))
_OUT = jax.ShapeDtypeStruct((NPAD, D), jnp.float32)

_tc_first = pl.pallas_call(
    _tc_first_body, grid=(GRID,),
    in_specs=[_ROWS, _MAT, _DEG, _DEG], out_specs=_ROWS, out_shape=_OUT)

_tc_mid = pl.pallas_call(
    _tc_mid_body, grid=(GRID,),
    in_specs=[_HROWS, _HROWS, _HROWS, _HROWS, _DEG, _DEG, _VEC, _MAT],
    out_specs=_ROWS, out_shape=_OUT)

_tc_last = pl.pallas_call(
    _tc_last_body, grid=(GRID,),
    in_specs=[_HROWS, _HROWS, _HROWS, _HROWS, _DEG, _DEG, _VEC, _MAT, _VEC],
    out_specs=_ROWS, out_shape=_OUT)


def kernel(x, edge_index, W0, b0, W1, b1, W2, b2, fcW, fcb):
    ei = edge_index.astype(jnp.int32)
    loop = jnp.arange(N, dtype=jnp.int32)
    npad_e = E_PAD - E_TOT
    pad_src = jnp.arange(npad_e, dtype=jnp.int32) % N
    src = jnp.concatenate([ei[0], loop, pad_src])
    pad_dst = N + (jnp.arange(npad_e, dtype=jnp.int32) % (NPAD - N))
    dst = jnp.concatenate([ei[1], loop, pad_dst])
    src0 = src * 2
    src1 = src0 + 1
    xp = jnp.pad(x, ((0, NPAD - N), (0, 0)))

    degw = _deg_kernel(dst)
    d0, d1 = degw[0], degw[1]

    def halves(g):
        return g.reshape(2 * NPAD, HALF)

    src0_3 = src0.reshape(NW, NCHUNKS, CHUNK)
    src1_3 = src1.reshape(NW, NCHUNKS, CHUNK)
    dst3 = dst.reshape(NW, NCHUNKS, CHUNK)

    g0 = _tc_first(xp, W0, d0, d1)
    a0 = _edge_kernel(src0_3, src1_3, dst3, halves(g0))
    g1 = _tc_mid(a0[0, 0], a0[1, 0], a0[0, 1], a0[1, 1], d0, d1,
                 b0.reshape(1, D), W1)
    a1 = _edge_kernel(src0_3, src1_3, dst3, halves(g1))
    g2 = _tc_mid(a1[0, 0], a1[1, 0], a1[0, 1], a1[1, 1], d0, d1,
                 b1.reshape(1, D), W2)
    a2 = _edge_kernel(src0_3, src1_3, dst3, halves(g2))
    out = _tc_last(a2[0, 0], a2[1, 0], a2[0, 1], a2[1, 1], d0, d1,
                   b2.reshape(1, D), fcW, fcb.reshape(1, D))
    return out[:N]


# speedup vs baseline: 3.9219x; 1.0727x over previous
import functools

import jax
import jax.numpy as jnp
from jax import lax
from jax.experimental import pallas as pl
from jax.experimental.pallas import tpu as pltpu
from jax.experimental.pallas import tpu_sc as plsc

N = 10000
NPAD = 10112
D = 128
NC, NS, L = 2, 16, 16
NW = NC * NS
CHUNK = 128
E_TOT = 320000 + N
EPW = 10752
E_PAD = NW * EPW
NCHUNKS = EPW // CHUNK
RPT = NPAD // NS
ZR = RPT // 2

_MESH = dict(core_axis_name="c", subcore_axis_name="s")


@functools.partial(
    pl.kernel,
    out_type=jax.ShapeDtypeStruct((NC, NPAD, L), jnp.float32),
    mesh=plsc.VectorSubcoreMesh(**_MESH),
    scratch_types=[
        pltpu.VMEM((NCHUNKS, CHUNK), jnp.int32),
        pltpu.VMEM((CHUNK, L), jnp.float32),
        pltpu.VMEM((ZR, L), jnp.float32),
        pltpu.VMEM_SHARED((NPAD, L), jnp.float32),
        pltpu.SemaphoreType.DMA,
    ],
    compiler_params=pltpu.CompilerParams(use_tc_tiling_on_sc=False),
)
def _deg_kernel(dst_hbm, out_hbm, di_all, ones_v, zero_v, acc, sem):
    c = lax.axis_index("c")
    s = lax.axis_index("s")
    w = c * NS + s

    def fill(i, _):
        ones_v[i] = jnp.ones((L,), jnp.float32)
        return 0

    lax.fori_loop(0, CHUNK, fill, 0)

    def zfill(i, _):
        zero_v[i] = jnp.zeros((L,), jnp.float32)
        return 0

    lax.fori_loop(0, ZR, zfill, 0)
    pltpu.sync_copy(dst_hbm.at[w], di_all)
    pltpu.sync_copy(zero_v, acc.at[pl.ds(s * RPT, ZR)])
    pltpu.sync_copy(zero_v, acc.at[pl.ds(s * RPT + ZR, ZR)])
    plsc.subcore_barrier()

    @pl.loop(0, NCHUNKS, step=12)
    def group(t0):
        for j in range(12):
            pltpu.async_copy(ones_v, acc.at[di_all.at[t0 + j]], sem,
                             add=True)
        for j in range(12):
            pltpu.make_async_copy(ones_v, acc.at[di_all.at[t0]], sem).wait()

    plsc.subcore_barrier()
    pltpu.sync_copy(acc.at[pl.ds(s * RPT, RPT)],
                    out_hbm.at[c, pl.ds(s * RPT, RPT)])


HALF = D // 2


@functools.partial(
    pl.kernel,
    out_type=jax.ShapeDtypeStruct((NC, 2, NPAD, HALF), jnp.float32),
    mesh=plsc.VectorSubcoreMesh(**_MESH),
    scratch_types=[
        pltpu.VMEM((NCHUNKS, CHUNK), jnp.int32),
        pltpu.VMEM((NCHUNKS, CHUNK), jnp.int32),
        [pltpu.VMEM((CHUNK, HALF), jnp.float32) for _ in range(4)],
        pltpu.VMEM((ZR, HALF), jnp.float32),
        pltpu.VMEM_SHARED((NPAD, HALF), jnp.float32),
        [pltpu.SemaphoreType.DMA for _ in range(4)],
        [pltpu.SemaphoreType.DMA for _ in range(4)],
    ],
    compiler_params=pltpu.CompilerParams(use_tc_tiling_on_sc=False),
)
def _edge_kernel(src0_hbm, src1_hbm, dst_hbm, gv_hbm, out_hbm,
                 si_all, di_all, rows, zero_v, acc, gsems, ssems):
    c = lax.axis_index("c")
    s = lax.axis_index("s")
    w = c * NS + s

    def zfill(i, _):
        zero_v[i // 4, pl.ds((i % 4) * L, L)] = jnp.zeros((L,), jnp.float32)
        return 0

    lax.fori_loop(0, ZR * (HALF // L), zfill, 0)
    pltpu.sync_copy(dst_hbm.at[w], di_all)

    for h, src_hbm in ((0, src0_hbm), (1, src1_hbm)):
        pltpu.sync_copy(zero_v, acc.at[pl.ds(s * RPT, ZR)])
        pltpu.sync_copy(zero_v, acc.at[pl.ds(s * RPT + ZR, ZR)])
        pltpu.sync_copy(src_hbm.at[w], si_all)
        plsc.subcore_barrier()

        pltpu.async_copy(gv_hbm.at[si_all.at[0]], rows[0], gsems[0])
        pltpu.async_copy(gv_hbm.at[si_all.at[1]], rows[1], gsems[1])

        @pl.loop(0, NCHUNKS, step=4)
        def chunk_quad(t0):
            for b in range(4):
                t = t0 + b
                nb = (b + 2) % 4
                pltpu.make_async_copy(gv_hbm.at[si_all.at[t]], rows[b],
                                      gsems[b]).wait()
                pltpu.async_copy(rows[b], acc.at[di_all.at[t]], ssems[b],
                                 add=True)

                @pl.when((t >= 2) & (t + 2 < NCHUNKS))
                def _():
                    pltpu.make_async_copy(rows[nb], acc.at[di_all.at[t]],
                                          ssems[nb]).wait()

                @pl.when(t + 2 < NCHUNKS)
                def _():
                    pltpu.async_copy(gv_hbm.at[si_all.at[t + 2]], rows[nb],
                                     gsems[nb])

        for b in range(4):
            pltpu.make_async_copy(rows[b], acc.at[di_all.at[0]],
                                  ssems[b]).wait()
        plsc.subcore_barrier()
        pltpu.sync_copy(acc.at[pl.ds(s * RPT, RPT)],
                        out_hbm.at[c, h, pl.ds(s * RPT, RPT)])


BR = 2528
GRID = NPAD // BR


def _dinv_block(d0_ref, d1_ref):
    deg = d0_ref[:, :1] + d1_ref[:, :1]
    return jnp.where(deg > 0, lax.rsqrt(deg), 0.0)


def _tc_first_body(x_ref, w_ref, d0_ref, d1_ref, g_ref):
    dinv = _dinv_block(d0_ref, d1_ref)
    g_ref[...] = dinv * jnp.dot(x_ref[...], w_ref[...],
                                preferred_element_type=jnp.float32)


def _combine(p00, p10, p01, p11, d0_ref, d1_ref, b_ref):
    dinv = _dinv_block(d0_ref, d1_ref)
    h0 = jnp.maximum(dinv * (p00[...] + p10[...]) + b_ref[:, :HALF], 0.0)
    h1 = jnp.maximum(dinv * (p01[...] + p11[...]) + b_ref[:, HALF:], 0.0)
    return dinv, jnp.concatenate([h0, h1], axis=1)


def _tc_mid_body(p00, p10, p01, p11, d0_ref, d1_ref, b_ref, w_ref, g_ref):
    dinv, h = _combine(p00, p10, p01, p11, d0_ref, d1_ref, b_ref)
    g_ref[...] = dinv * jnp.dot(h, w_ref[...],
                                preferred_element_type=jnp.float32)


def _tc_last_body(p00, p10, p01, p11, d0_ref, d1_ref, b_ref, w_ref, bb_ref,
                  o_ref):
    _, h = _combine(p00, p10, p01, p11, d0_ref, d1_ref, b_ref)
    o_ref[...] = jnp.dot(h, w_ref[...],
                         preferred_element_type=jnp.float32) + bb_ref[...]


_ROWS = pl.BlockSpec((BR, D), lambda i: (i, 0))
_HROWS = pl.BlockSpec((BR, HALF), lambda i: (i, 0))
_MAT = pl.BlockSpec((D, D), lambda i: (0, 0))
_DEG = pl.BlockSpec((BR, L), lambda i: (i, 0))
_VEC = pl.BlockSpec((1, D), lambda i: (0, 0))
_OUT = jax.ShapeDtypeStruct((NPAD, D), jnp.float32)

_tc_first = pl.pallas_call(
    _tc_first_body, grid=(GRID,),
    in_specs=[_ROWS, _MAT, _DEG, _DEG], out_specs=_ROWS, out_shape=_OUT)

_tc_mid = pl.pallas_call(
    _tc_mid_body, grid=(GRID,),
    in_specs=[_HROWS, _HROWS, _HROWS, _HROWS, _DEG, _DEG, _VEC, _MAT],
    out_specs=_ROWS, out_shape=_OUT)

_tc_last = pl.pallas_call(
    _tc_last_body, grid=(GRID,),
    in_specs=[_HROWS, _HROWS, _HROWS, _HROWS, _DEG, _DEG, _VEC, _MAT, _VEC],
    out_specs=_ROWS, out_shape=_OUT)


def kernel(x, edge_index, W0, b0, W1, b1, W2, b2, fcW, fcb):
    ei = edge_index.astype(jnp.int32)
    loop = jnp.arange(N, dtype=jnp.int32)
    npad_e = E_PAD - E_TOT
    pad_src = jnp.arange(npad_e, dtype=jnp.int32) % N
    src = jnp.concatenate([ei[0], loop, pad_src])
    pad_dst = N + (jnp.arange(npad_e, dtype=jnp.int32) % (NPAD - N))
    dst = jnp.concatenate([ei[1], loop, pad_dst])
    src0 = src * 2
    src1 = src0 + 1
    xp = jnp.pad(x, ((0, NPAD - N), (0, 0)))

    src0_3 = src0.reshape(NW, NCHUNKS, CHUNK)
    src1_3 = src1.reshape(NW, NCHUNKS, CHUNK)
    dst3 = dst.reshape(NW, NCHUNKS, CHUNK)

    degw = _deg_kernel(dst3)
    d0, d1 = degw[0], degw[1]

    def halves(g):
        return g.reshape(2 * NPAD, HALF)

    g0 = _tc_first(xp, W0, d0, d1)
    a0 = _edge_kernel(src0_3, src1_3, dst3, halves(g0))
    g1 = _tc_mid(a0[0, 0], a0[1, 0], a0[0, 1], a0[1, 1], d0, d1,
                 b0.reshape(1, D), W1)
    a1 = _edge_kernel(src0_3, src1_3, dst3, halves(g1))
    g2 = _tc_mid(a1[0, 0], a1[1, 0], a1[0, 1], a1[1, 1], d0, d1,
                 b1.reshape(1, D), W2)
    a2 = _edge_kernel(src0_3, src1_3, dst3, halves(g2))
    out = _tc_last(a2[0, 0], a2[1, 0], a2[0, 1], a2[1, 1], d0, d1,
                   b2.reshape(1, D), fcW, fcb.reshape(1, D))
    return out[:N]
